# Initial kernel scaffold; baseline (speedup 1.0000x reference)
import jax
import jax.numpy as jnp
from jax.experimental import pallas as pl


def kernel(x, edge_index, W1, att_src1, att_dst1, b1, W2, att_src2, att_dst2, b2):
    raise NotImplementedError("write your pallas kernel here")


# speedup vs baseline: 51.2310x; 51.2310x over previous
import dataclasses
import functools

import jax
import jax.numpy as jnp
from jax import lax
from jax.experimental import pallas as pl
from jax.experimental.pallas import tpu as pltpu
from jax.experimental.pallas import tpu_sc as plsc

N = 10000
E = 320000
F_IN = 128
H1 = 8
C1 = 8
D1 = H1 * C1
C2 = 7

NC = 2
NS = 16
NW = NC * NS
EPW = E // NW
EB = 80
NB = EPW // EB
RPT = 624

AW1 = 80
AW2 = 16

_LEAK = 0.2
_BIG = 1e30


def _sc_compiler_params():
    cp = pltpu.CompilerParams()
    fields = pltpu.CompilerParams.__dataclass_fields__
    if "needs_layout_passes" in fields:
        cp = dataclasses.replace(cp, needs_layout_passes=False)
    if "use_tc_tiling_on_sc" in fields:
        cp = dataclasses.replace(cp, use_tc_tiling_on_sc=False)
    return cp


def _tc_a_body(x_ref, w1_ref, as_ref, ad_ref, th_ref, tasd_ref, g_ref, gv_ref):
    i = pl.program_id(0)
    ng = pl.num_programs(0)
    h = jnp.dot(x_ref[...], w1_ref[...], preferred_element_type=jnp.float32)
    th_ref[...] = h
    a_s = jnp.dot(h, as_ref[...], preferred_element_type=jnp.float32)
    a_d = jnp.dot(h, ad_ref[...], preferred_element_type=jnp.float32)
    tasd_ref[...] = jnp.concatenate([a_s, a_d], axis=1)
    bmax = jnp.concatenate(
        [jnp.max(a_s, axis=0, keepdims=True), jnp.max(a_d, axis=0, keepdims=True)],
        axis=1)

    @pl.when(i == 0)
    def _():
        g_ref[...] = bmax

    @pl.when(i > 0)
    def _():
        g_ref[...] = jnp.maximum(g_ref[...], bmax)

    @pl.when(i == ng - 1)
    def _():
        gp = g_ref[...]
        g8 = gp[:, :H1] + gp[:, H1:]
        gv_ref[...] = jnp.concatenate(
            [g8, jnp.full((1, 8), _BIG, jnp.float32)], axis=1)


def _tc_a(x, w1, a_s_fold, a_d_fold, rows):
    grid = (N // rows,)
    return pl.pallas_call(
        _tc_a_body,
        grid=grid,
        in_specs=[
            pl.BlockSpec((rows, F_IN), lambda i: (i, 0)),
            pl.BlockSpec((F_IN, D1), lambda i: (0, 0)),
            pl.BlockSpec((D1, H1), lambda i: (0, 0)),
            pl.BlockSpec((D1, H1), lambda i: (0, 0)),
        ],
        out_specs=[
            pl.BlockSpec((rows, D1), lambda i: (i, 0)),
            pl.BlockSpec((rows, 2 * H1), lambda i: (i, 0)),
            pl.BlockSpec((1, 16), lambda i: (0, 0)),
            pl.BlockSpec((1, 16), lambda i: (0, 0)),
        ],
        out_shape=[
            jax.ShapeDtypeStruct((N, D1), jnp.float32),
            jax.ShapeDtypeStruct((N, 2 * H1), jnp.float32),
            jax.ShapeDtypeStruct((1, 16), jnp.float32),
            jax.ShapeDtypeStruct((1, 16), jnp.float32),
        ],
    )(x, w1, a_s_fold, a_d_fold)


def _sc1_body(src_hbm, dst_hbm, th_hbm, tasd_hbm, z80_hbm, gv_hbm, out_hbm,
              th_sp, tasd_sp, acc_sp, sidx, didx, sbuf, dbuf, hbuf, msg, gbuf):
    cid = lax.axis_index("c")
    sid = lax.axis_index("s")
    wid = cid * NS + sid
    r0 = sid * RPT
    pltpu.sync_copy(th_hbm.at[pl.ds(r0, RPT)], th_sp.at[pl.ds(r0, RPT)])
    pltpu.sync_copy(tasd_hbm.at[pl.ds(r0, RPT)], tasd_sp.at[pl.ds(r0, RPT)])
    pltpu.sync_copy(z80_hbm.at[pl.ds(r0, RPT)], acc_sp.at[pl.ds(r0, RPT)])
    pltpu.sync_copy(gv_hbm, gbuf)

    @pl.when(sid == NS - 1)
    def _():
        rr = NS * RPT
        nr = N - rr
        pltpu.sync_copy(th_hbm.at[pl.ds(rr, nr)], th_sp.at[pl.ds(rr, nr)])
        pltpu.sync_copy(tasd_hbm.at[pl.ds(rr, nr)], tasd_sp.at[pl.ds(rr, nr)])
        pltpu.sync_copy(z80_hbm.at[pl.ds(rr, nr)], acc_sp.at[pl.ds(rr, nr)])

    plsc.subcore_barrier()

    gv = gbuf[...]
    iota = lax.iota(jnp.int32, 16)
    rot8 = (iota + 8) & 15
    ebase = wid * EPW

    @pl.loop(0, NB)
    def _blk(blk):
        off = ebase + blk * EB
        pltpu.sync_copy(src_hbm.at[pl.ds(off, EB)], sidx)
        pltpu.sync_copy(dst_hbm.at[pl.ds(off, EB)], didx)
        pltpu.sync_copy(tasd_sp.at[sidx], sbuf)
        pltpu.sync_copy(tasd_sp.at[didx], dbuf)
        pltpu.sync_copy(th_sp.at[sidx], hbuf)

        @pl.loop(0, EB)
        def _e(e):
            efull = jnp.full((16,), e, jnp.int32)
            u = sbuf[e, :]
            vg = plsc.load_gather(dbuf, [efull, rot8])
            t = u + vg
            lk = jnp.maximum(t, _LEAK * t)
            ex = jnp.exp(lk - gv)
            msg[e, pl.ds(D1, 16)] = ex
            for k in range(4):
                bidx = D1 + (iota >> 3) + 2 * k
                exb = plsc.load_gather(msg, [efull, bidx])
                msg[e, pl.ds(16 * k, 16)] = exb * hbuf[e, pl.ds(16 * k, 16)]

        pltpu.sync_copy(msg, acc_sp.at[didx], add=True)

    plsc.subcore_barrier()
    pltpu.sync_copy(acc_sp.at[pl.ds(r0, RPT)], out_hbm.at[cid, pl.ds(r0, RPT)])

    @pl.when(sid == NS - 1)
    def _():
        rr = NS * RPT
        nr = N - rr
        pltpu.sync_copy(acc_sp.at[pl.ds(rr, nr)], out_hbm.at[cid, pl.ds(rr, nr)])


def _sc1(src, dst, th, tasd, z80, gv):
    mesh = plsc.VectorSubcoreMesh(core_axis_name="c", subcore_axis_name="s")
    kern = functools.partial(
        pl.kernel,
        out_type=jax.ShapeDtypeStruct((NC, N, AW1), jnp.float32),
        mesh=mesh,
        compiler_params=_sc_compiler_params(),
        scratch_types=[
            pltpu.VMEM_SHARED((N, D1), jnp.float32),
            pltpu.VMEM_SHARED((N, 2 * H1), jnp.float32),
            pltpu.VMEM_SHARED((N, AW1), jnp.float32),
            pltpu.VMEM((EB,), jnp.int32),
            pltpu.VMEM((EB,), jnp.int32),
            pltpu.VMEM((EB, 16), jnp.float32),
            pltpu.VMEM((EB, 16), jnp.float32),
            pltpu.VMEM((EB, D1), jnp.float32),
            pltpu.VMEM((EB, AW1), jnp.float32),
            pltpu.VMEM((16,), jnp.float32),
        ],
    )(_sc1_body)
    return kern(src, dst, th, tasd, z80, gv)


def _tc_b_body(acc_ref, th_ref, tasd_ref, gv_ref, b1_ref, w2p_ref, p_ref,
               t2_ref, g2_ref, g2v_ref):
    i = pl.program_id(0)
    ng = pl.num_programs(0)
    a0 = acc_ref[0]
    a1 = acc_ref[1]
    msum = a0[:, :D1] + a1[:, :D1]
    dsum = a0[:, D1:D1 + H1] + a1[:, D1:D1 + H1]
    asd = tasd_ref[...]
    a_s = asd[:, :H1]
    a_d = asd[:, H1:]
    t = a_s + a_d
    lk = jnp.maximum(t, _LEAK * t)
    g8 = gv_ref[0, :H1]
    ex = jnp.exp(lk - g8[None, :])
    h1v = th_ref[...]
    p = p_ref[...]
    msg = msum + jnp.dot(ex, p, preferred_element_type=jnp.float32) * h1v
    den = jnp.dot(dsum + ex, p, preferred_element_type=jnp.float32)
    o1 = msg / (den + 1e-16) + b1_ref[...]
    h1r = jnp.maximum(o1, 0.0)
    t2 = jnp.dot(h1r, w2p_ref[...], preferred_element_type=jnp.float32)
    t2_ref[...] = t2
    bmax = jnp.max(t2, axis=0, keepdims=True)

    @pl.when(i == 0)
    def _():
        g2_ref[...] = bmax

    @pl.when(i > 0)
    def _():
        g2_ref[...] = jnp.maximum(g2_ref[...], bmax)

    @pl.when(i == ng - 1)
    def _():
        gp = g2_ref[...]
        g2 = gp[0, C2] + gp[0, C2 + 1]
        g2v_ref[...] = jnp.full((1, 16), g2, jnp.float32)


def _tc_b(acc, th, tasd, gv, b1r, w2p, p, rows):
    grid = (N // rows,)
    return pl.pallas_call(
        _tc_b_body,
        grid=grid,
        in_specs=[
            pl.BlockSpec((NC, rows, AW1), lambda i: (0, i, 0)),
            pl.BlockSpec((rows, D1), lambda i: (i, 0)),
            pl.BlockSpec((rows, 2 * H1), lambda i: (i, 0)),
            pl.BlockSpec((1, 16), lambda i: (0, 0)),
            pl.BlockSpec((1, D1), lambda i: (0, 0)),
            pl.BlockSpec((D1, 16), lambda i: (0, 0)),
            pl.BlockSpec((H1, D1), lambda i: (0, 0)),
        ],
        out_specs=[
            pl.BlockSpec((rows, 16), lambda i: (i, 0)),
            pl.BlockSpec((1, 16), lambda i: (0, 0)),
            pl.BlockSpec((1, 16), lambda i: (0, 0)),
        ],
        out_shape=[
            jax.ShapeDtypeStruct((N, 16), jnp.float32),
            jax.ShapeDtypeStruct((1, 16), jnp.float32),
            jax.ShapeDtypeStruct((1, 16), jnp.float32),
        ],
    )(acc, th, tasd, gv, b1r, w2p, p)


def _sc2_body(src_hbm, dst_hbm, t2_hbm, z16_hbm, g2v_hbm, out_hbm,
              t2_sp, acc_sp, sidx, didx, ubuf, vbuf, msg, gbuf):
    cid = lax.axis_index("c")
    sid = lax.axis_index("s")
    wid = cid * NS + sid
    r0 = sid * RPT
    pltpu.sync_copy(t2_hbm.at[pl.ds(r0, RPT)], t2_sp.at[pl.ds(r0, RPT)])
    pltpu.sync_copy(z16_hbm.at[pl.ds(r0, RPT)], acc_sp.at[pl.ds(r0, RPT)])
    pltpu.sync_copy(g2v_hbm, gbuf)

    @pl.when(sid == NS - 1)
    def _():
        rr = NS * RPT
        nr = N - rr
        pltpu.sync_copy(t2_hbm.at[pl.ds(rr, nr)], t2_sp.at[pl.ds(rr, nr)])
        pltpu.sync_copy(z16_hbm.at[pl.ds(rr, nr)], acc_sp.at[pl.ds(rr, nr)])

    plsc.subcore_barrier()

    gv = gbuf[...]
    iota = lax.iota(jnp.int32, 16)
    i7 = jnp.full((16,), C2, jnp.int32)
    i8 = jnp.full((16,), C2 + 1, jnp.int32)
    ebase = wid * EPW

    @pl.loop(0, NB)
    def _blk(blk):
        off = ebase + blk * EB
        pltpu.sync_copy(src_hbm.at[pl.ds(off, EB)], sidx)
        pltpu.sync_copy(dst_hbm.at[pl.ds(off, EB)], didx)
        pltpu.sync_copy(t2_sp.at[sidx], ubuf)
        pltpu.sync_copy(t2_sp.at[didx], vbuf)

        @pl.loop(0, EB)
        def _e(e):
            efull = jnp.full((16,), e, jnp.int32)
            u = ubuf[e, :]
            bu = plsc.load_gather(ubuf, [efull, i7])
            bv = plsc.load_gather(vbuf, [efull, i8])
            t = bu + bv
            lk = jnp.maximum(t, _LEAK * t)
            ex = jnp.exp(lk - gv)
            m = jnp.where(iota < C2, ex * u,
                          jnp.where(iota == C2, ex, jnp.zeros_like(ex)))
            msg[e, :] = m

        pltpu.sync_copy(msg, acc_sp.at[didx], add=True)

    plsc.subcore_barrier()
    pltpu.sync_copy(acc_sp.at[pl.ds(r0, RPT)], out_hbm.at[cid, pl.ds(r0, RPT)])

    @pl.when(sid == NS - 1)
    def _():
        rr = NS * RPT
        nr = N - rr
        pltpu.sync_copy(acc_sp.at[pl.ds(rr, nr)], out_hbm.at[cid, pl.ds(rr, nr)])


def _sc2(src, dst, t2, z16, g2v):
    mesh = plsc.VectorSubcoreMesh(core_axis_name="c", subcore_axis_name="s")
    kern = functools.partial(
        pl.kernel,
        out_type=jax.ShapeDtypeStruct((NC, N, AW2), jnp.float32),
        mesh=mesh,
        compiler_params=_sc_compiler_params(),
        scratch_types=[
            pltpu.VMEM_SHARED((N, 16), jnp.float32),
            pltpu.VMEM_SHARED((N, AW2), jnp.float32),
            pltpu.VMEM((EB,), jnp.int32),
            pltpu.VMEM((EB,), jnp.int32),
            pltpu.VMEM((EB, 16), jnp.float32),
            pltpu.VMEM((EB, 16), jnp.float32),
            pltpu.VMEM((EB, AW2), jnp.float32),
            pltpu.VMEM((16,), jnp.float32),
        ],
    )(_sc2_body)
    return kern(src, dst, t2, z16, g2v)


def _tc_c_body(acc_ref, t2_ref, g2v_ref, b2_ref, out_ref):
    a = acc_ref[0] + acc_ref[1]
    t2 = t2_ref[...]
    rows = t2.shape[0]
    a_s2 = t2[:, C2:C2 + 1]
    a_d2 = t2[:, C2 + 1:C2 + 2]
    t = a_s2 + a_d2
    lk = jnp.maximum(t, _LEAK * t)
    g2 = g2v_ref[0, 0]
    ex = jnp.exp(lk - g2)
    lane = lax.broadcasted_iota(jnp.int32, (rows, 16), 1)
    contrib = jnp.where(lane < C2, ex * t2,
                        jnp.where(lane == C2, jnp.broadcast_to(ex, (rows, 16)),
                                  0.0))
    tot = a + contrib
    den = tot[:, C2:C2 + 1]
    o = tot / (den + 1e-16) + b2_ref[...]
    om = jnp.where(lane < C2, o, -1e30)
    m = jnp.max(om, axis=1, keepdims=True)
    z = jnp.exp(om - m)
    s = jnp.sum(z, axis=1, keepdims=True)
    ls = om - m - jnp.log(s)
    out_ref[...] = ls[:, :C2]


def _tc_c(acc2, t2, g2v, b2r, rows):
    grid = (N // rows,)
    return pl.pallas_call(
        _tc_c_body,
        grid=grid,
        in_specs=[
            pl.BlockSpec((NC, rows, AW2), lambda i: (0, i, 0)),
            pl.BlockSpec((rows, 16), lambda i: (i, 0)),
            pl.BlockSpec((1, 16), lambda i: (0, 0)),
            pl.BlockSpec((1, 16), lambda i: (0, 0)),
        ],
        out_specs=pl.BlockSpec((rows, C2), lambda i: (i, 0)),
        out_shape=jax.ShapeDtypeStruct((N, C2), jnp.float32),
    )(acc2, t2, g2v, b2r)


def kernel(x, edge_index, W1, att_src1, att_dst1, b1, W2, att_src2, att_dst2, b2):
    src = edge_index[0].astype(jnp.int32)
    dst = edge_index[1].astype(jnp.int32)

    eye8 = jnp.eye(H1, dtype=jnp.float32)
    a_s_fold = (att_src1[:, :, None] * eye8[:, None, :]).reshape(D1, H1)
    a_d_fold = (att_dst1[:, :, None] * eye8[:, None, :]).reshape(D1, H1)
    p = jnp.repeat(eye8, C1, axis=1)
    vs = W2 @ att_src2[0]
    vd = W2 @ att_dst2[0]
    w2p = jnp.concatenate(
        [W2, vs[:, None], vd[:, None], jnp.zeros((D1, 16 - C2 - 2), jnp.float32)],
        axis=1)
    b1r = b1.reshape(1, D1)
    b2r = jnp.concatenate([b2, jnp.zeros((16 - C2,), jnp.float32)]).reshape(1, 16)

    z80 = jnp.zeros((N, AW1), jnp.float32)
    z16 = jnp.zeros((N, AW2), jnp.float32)

    th, tasd, _gpair, gv = _tc_a(x, W1, a_s_fold, a_d_fold, rows=1000)
    gv1 = gv.reshape(16)
    acc = _sc1(src, dst, th, tasd, z80, gv1)
    t2, _g2pair, g2v = _tc_b(acc, th, tasd, gv, b1r, w2p, p, rows=1000)
    acc2 = _sc2(src, dst, t2, z16, g2v.reshape(16))
    out = _tc_c(acc2, t2, g2v, b2r, rows=1000)
    return out
